# baseline (speedup 1.0000x reference)
import functools

import jax
import jax.numpy as jnp
from jax import lax
from jax.experimental import pallas as pl
from jax.experimental.pallas import tpu as pltpu
from jax.experimental.pallas import tpu_sc as plsc

B = 16384
L = 200
DIM = 32
N = B * L

NC = 2
NS = 16
NW = NC * NS
BW = B // NW

OUT_FLAT = L * 4 * 128 * 8 * 128

_mesh = plsc.VectorSubcoreMesh(core_axis_name="c", subcore_axis_name="s")


@functools.partial(
    pl.kernel,
    mesh=_mesh,
    out_type=jax.ShapeDtypeStruct((OUT_FLAT,), jnp.float32),
    scratch_types=[
        pltpu.VMEM((BW,), jnp.int32),
        pltpu.VMEM((BW,), jnp.int32),
        pltpu.VMEM((BW, DIM), jnp.float32),
        pltpu.VMEM((BW, DIM), jnp.float32),
        pltpu.VMEM((4, 4096), jnp.float32),
        pltpu.VMEM((4, 4096), jnp.float32),
        pltpu.SemaphoreType.DMA,
        pltpu.SemaphoreType.DMA,
        pltpu.SemaphoreType.DMA,
        pltpu.SemaphoreType.DMA,
        pltpu.SemaphoreType.DMA,
        pltpu.SemaphoreType.DMA,
    ],
    compiler_params=pltpu.CompilerParams(
        use_tc_tiling_on_sc=False, needs_layout_passes=False),
)
def _gather_kernel(idx_hbm, table_hbm, out_hbm,
                   idx_v0, idx_v1, rows_v0, rows_v1, t_v0, t_v1,
                   sem_i0, sem_i1, sem_g0, sem_g1, sem_o0, sem_o1):
    w = lax.axis_index("s") * NC + lax.axis_index("c")
    idx_v = [idx_v0, idx_v1]
    rows_v = [rows_v0, rows_v1]
    t_v = [t_v0, t_v1]
    sems_i = [sem_i0, sem_i1]
    sems_g = [sem_g0, sem_g1]
    sems_o = [sem_o0, sem_o1]

    iota16 = lax.iota(jnp.int32, 16)

    def idx_copy(l, p):
        return pltpu.make_async_copy(
            idx_hbm.at[pl.ds(l * B + w * BW, BW)], idx_v[p], sems_i[p])

    def gather_copy(p):
        return pltpu.make_async_copy(
            table_hbm.at[idx_v[p]], rows_v[p], sems_g[p])

    def out_copy(l, r, p):
        off = ((l * 4 + r) * 128 + 4 * w) * 1024
        return pltpu.make_async_copy(
            t_v[p].at[r], out_hbm.at[pl.ds(off, 4096)], sems_o[p])

    idx_copy(0, 0).start()
    idx_copy(1, 1).start()
    idx_copy(0, 0).wait()
    gather_copy(0).start()

    def body(l, carry):
        def on(p):
            @pl.when(p == lax.rem(l, 2))
            def _():
                gather_copy(p).wait()

                @pl.when(l < L - 1)
                def _():
                    idx_copy(l + 1, 1 - p).wait()
                    gather_copy(1 - p).start()

                @pl.when(l < L - 2)
                def _():
                    idx_copy(l + 2, p).start()

                @pl.when(l >= 2)
                def _():
                    for r in range(4):
                        out_copy(l - 2, r, p).wait()

                def tile_rc(rc, carry2):
                    r = rc // 4
                    cp = rc - r * 4
                    rowbase = cp * 128
                    for bb in range(8):
                        rvec = iota16 + (rowbase + bb * 16)
                        for dd in range(8):
                            cvec = jnp.full((16,), 8 * r + dd, jnp.int32)
                            v = plsc.load_gather(rows_v[p], [rvec, cvec])
                            t_v[p][r, pl.ds(cp * 1024 + dd * 128 + bb * 16,
                                            16)] = v
                    return carry2

                lax.fori_loop(0, 16, tile_rc, 0)

                for r in range(4):
                    out_copy(l, r, p).start()
        on(0)
        on(1)
        return carry

    lax.fori_loop(0, L, body, 0)

    for l in (L - 2, L - 1):
        for r in range(4):
            out_copy(l, r, l % 2).wait()


def kernel(x, table):
    flat = x.T.reshape(N).astype(jnp.int32)
    out_flat = _gather_kernel(flat, table)
    out5 = out_flat.reshape(L, 4, 128, 8, 128)
    return out5.transpose(2, 4, 0, 1, 3).reshape(B, L, DIM)


# speedup vs baseline: 2.9127x; 2.9127x over previous
import functools

import jax
import jax.numpy as jnp
from jax import lax
from jax.experimental import pallas as pl
from jax.experimental.pallas import tpu as pltpu
from jax.experimental.pallas import tpu_sc as plsc

B = 16384
L = 200
DIM = 32
N = B * L

NC = 2
NS = 16
NW = NC * NS
BW = B // NW

TS = 129
OUT_ROWS = L * 4 * 128 * 8

_mesh = plsc.VectorSubcoreMesh(core_axis_name="c", subcore_axis_name="s")


@functools.partial(
    pl.kernel,
    mesh=_mesh,
    out_type=jax.ShapeDtypeStruct((OUT_ROWS, 128), jnp.float32),
    scratch_types=[
        pltpu.VMEM((BW,), jnp.int32),
        pltpu.VMEM((BW,), jnp.int32),
        pltpu.VMEM((BW, DIM), jnp.float32),
        pltpu.VMEM((BW, DIM), jnp.float32),
        pltpu.VMEM((128, TS), jnp.float32),
        pltpu.VMEM((128, TS), jnp.float32),
        pltpu.SemaphoreType.DMA,
        pltpu.SemaphoreType.DMA,
        pltpu.SemaphoreType.DMA,
        pltpu.SemaphoreType.DMA,
        pltpu.SemaphoreType.DMA,
        pltpu.SemaphoreType.DMA,
    ],
    compiler_params=pltpu.CompilerParams(
        use_tc_tiling_on_sc=False, needs_layout_passes=False,
        disable_bounds_checks=True),
)
def _gather_kernel(idx_hbm, table_hbm, out_hbm,
                   idx_v0, idx_v1, rows_v0, rows_v1, t_v0, t_v1,
                   sem_i0, sem_i1, sem_g0, sem_g1, sem_o0, sem_o1):
    w = lax.axis_index("s") * NC + lax.axis_index("c")
    idx_v = [idx_v0, idx_v1]
    rows_v = [rows_v0, rows_v1]
    t_v = [t_v0, t_v1]
    sems_i = [sem_i0, sem_i1]
    sems_g = [sem_g0, sem_g1]
    sems_o = [sem_o0, sem_o1]

    iota16 = lax.iota(jnp.int32, 16)

    def idx_copy(l, p):
        return pltpu.make_async_copy(
            idx_hbm.at[pl.ds(l * B + w * BW, BW)], idx_v[p], sems_i[p])

    def gather_copy(p):
        return pltpu.make_async_copy(
            table_hbm.at[idx_v[p]], rows_v[p], sems_g[p])

    def out_copy(l, r, cp, p):
        row0 = ((l * 4 + r) * 128 + 4 * w + cp) * 8
        return pltpu.make_async_copy(
            t_v[p].at[pl.ds(cp * 32 + 8 * r, 8), pl.ds(0, 128)],
            out_hbm.at[pl.ds(row0, 8), :], sems_o[p])

    idx_copy(0, 0).start()
    idx_copy(1, 1).start()
    idx_copy(0, 0).wait()
    gather_copy(0).start()

    def transpose(p):
        def blk(jo, carry2):
            rows_a = iota16 + ((jo >> 3) << 5)
            rows_b = rows_a + 16
            colbase = (jo & 7) << 4
            for jj in range(16):
                j = jo * 16 + jj
                colv = jnp.full((16,), colbase + jj, jnp.int32)
                lo = rows_v[p][j, pl.ds(0, 16)]
                hi = rows_v[p][j, pl.ds(16, 16)]
                plsc.store_scatter(t_v[p], [rows_a, colv], lo)
                plsc.store_scatter(t_v[p], [rows_b, colv], hi)
            return carry2

        lax.fori_loop(0, BW // 16, blk, 0)

    def body(lo_i, carry):
        def stage(l, p, cond_fire, cond_prefetch, cond_prev):
            gather_copy(p).wait()

            def fire():
                idx_copy(l + 1, 1 - p).wait()
                gather_copy(1 - p).start()

                @pl.when(cond_prefetch)
                def _():
                    idx_copy(l + 2, p).start()

            if cond_fire is True:
                fire()
            else:
                pl.when(cond_fire)(fire)

            @pl.when(cond_prev)
            def _():
                for r in range(4):
                    for cp in range(4):
                        out_copy(l - 2, r, cp, p).wait()

            transpose(p)

            for r in range(4):
                for cp in range(4):
                    out_copy(l, r, cp, p).start()

        l0 = 2 * lo_i
        half = L // 2
        last = lo_i < half - 1
        stage(l0, 0, True, last, lo_i > 0)
        stage(l0 + 1, 1, last, last, lo_i > 0)
        return carry

    lax.fori_loop(0, L // 2, body, 0)

    for l in (L - 2, L - 1):
        for r in range(4):
            for cp in range(4):
                out_copy(l, r, cp, l % 2).wait()


def kernel(x, table):
    flat = x.T.reshape(N).astype(jnp.int32)
    out2 = _gather_kernel(flat, table)
    out5 = out2.reshape(L, 4, 128, 8, 128)
    return out5.transpose(2, 4, 0, 1, 3).reshape(B, L, DIM)
